# baseline (device time: 1003436 ns/iter reference)
import jax
import jax.numpy as jnp
from jax import lax
from jax.experimental import pallas as pl
from jax.experimental.pallas import tpu as pltpu

ROW_CHUNKS = [512] * 31 + [128] * 4
STAGE_ROWS = 2048


def kernel(x):
    m, n = x.shape
    half = m // 2
    n_chunks = len(ROW_CHUNKS)
    offsets = [sum(ROW_CHUNKS[:i]) for i in range(n_chunks)]
    n_stage = m // STAGE_ROWS

    def body(x_ref, out_ref, stage_vmem, ld_sems, st_sems,
             x_send, x_recv, y_send, y_recv):
        my_x = lax.axis_index("x")
        my_y = lax.axis_index("y")
        other_x = 1 - my_x
        base = my_x * m

        x_rdmas = []
        for c in range(n_chunks):
            row = my_y * half + offsets[c]
            ch = ROW_CHUNKS[c]
            r = pltpu.make_async_remote_copy(
                src_ref=x_ref.at[pl.ds(row, ch), :],
                dst_ref=out_ref.at[pl.ds(base + row, ch), :],
                send_sem=x_send.at[c],
                recv_sem=x_recv.at[c],
                device_id=(other_x, my_y),
                device_id_type=pl.DeviceIdType.MESH,
            )
            r.start()
            x_rdmas.append(r)

        stage_stores = [None, None]
        for c in range(n_stage):
            slot = c % 2
            if stage_stores[slot] is not None:
                stage_stores[slot].wait()
            ld = pltpu.make_async_copy(
                x_ref.at[pl.ds(c * STAGE_ROWS, STAGE_ROWS), :],
                stage_vmem.at[slot],
                ld_sems.at[slot],
            )
            ld.start()
            ld.wait()
            st = pltpu.make_async_copy(
                stage_vmem.at[slot],
                out_ref.at[pl.ds(base + c * STAGE_ROWS, STAGE_ROWS), :],
                st_sems.at[slot],
            )
            st.start()
            stage_stores[slot] = st

        y_rdmas = []
        for c in range(n_chunks):
            x_rdmas[c].wait_recv()
            row = other_x * m + my_y * half + offsets[c]
            ch = ROW_CHUNKS[c]
            f = pltpu.make_async_remote_copy(
                src_ref=out_ref.at[pl.ds(row, ch), :],
                dst_ref=out_ref.at[pl.ds(row, ch), :],
                send_sem=y_send.at[c],
                recv_sem=y_recv.at[c],
                device_id=(my_x, 1 - my_y),
                device_id_type=pl.DeviceIdType.MESH,
            )
            f.start()
            y_rdmas.append(f)

        for c in range(n_chunks):
            y_rdmas[c].wait_recv()
        for c in range(n_chunks):
            x_rdmas[c].wait_send()
            y_rdmas[c].wait_send()
        stage_stores[0].wait()
        stage_stores[1].wait()

    return pl.pallas_call(
        body,
        out_shape=jax.ShapeDtypeStruct((2 * m, n), x.dtype),
        in_specs=[pl.BlockSpec(memory_space=pl.ANY)],
        out_specs=pl.BlockSpec(memory_space=pl.ANY),
        scratch_shapes=[
            pltpu.VMEM((2, STAGE_ROWS, n), x.dtype),
            pltpu.SemaphoreType.DMA((2,)),
            pltpu.SemaphoreType.DMA((2,)),
            pltpu.SemaphoreType.DMA((n_chunks,)),
            pltpu.SemaphoreType.DMA((n_chunks,)),
            pltpu.SemaphoreType.DMA((n_chunks,)),
            pltpu.SemaphoreType.DMA((n_chunks,)),
        ],
    )(x)


# device time: 924836 ns/iter; 1.0850x vs baseline; 1.0850x over previous
import jax
import jax.numpy as jnp
from jax import lax
from jax.experimental import pallas as pl
from jax.experimental.pallas import tpu as pltpu

ROW_CHUNKS = [128] * 4 + [512] * 30 + [128] * 4
STAGE_ROWS = 2048
N_SLOTS = 4


def kernel(x):
    m, n = x.shape
    half = m // 2
    n_chunks = len(ROW_CHUNKS)
    assert sum(ROW_CHUNKS) == half
    offsets = [sum(ROW_CHUNKS[:i]) for i in range(n_chunks)]
    n_stage = m // STAGE_ROWS

    def body(x_ref, out_ref, stage_vmem, ld_sems, st_sems,
             x_send, x_recv, y_send, y_recv):
        my_x = lax.axis_index("x")
        my_y = lax.axis_index("y")
        other_x = 1 - my_x
        base = my_x * m

        x_rdmas = []
        for c in range(n_chunks):
            row = my_y * half + offsets[c]
            ch = ROW_CHUNKS[c]
            r = pltpu.make_async_remote_copy(
                src_ref=x_ref.at[pl.ds(row, ch), :],
                dst_ref=out_ref.at[pl.ds(base + row, ch), :],
                send_sem=x_send.at[c],
                recv_sem=x_recv.at[c],
                device_id=(other_x, my_y),
                device_id_type=pl.DeviceIdType.MESH,
            )
            r.start()
            x_rdmas.append(r)

        lds = [None] * N_SLOTS
        sts = [None] * N_SLOTS
        issued = [0]

        def stage_issue():
            c = issued[0]
            if c >= n_stage:
                return False
            issued[0] = c + 1
            slot = c % N_SLOTS
            if sts[slot] is not None:
                sts[slot].wait()
            row = c * STAGE_ROWS
            ld = pltpu.make_async_copy(
                x_ref.at[pl.ds(row, STAGE_ROWS), :],
                stage_vmem.at[slot],
                ld_sems.at[slot],
            )
            ld.start()
            lds[slot] = ld
            prev = c - 1
            if prev >= 0:
                pslot = prev % N_SLOTS
                lds[pslot].wait()
                prow = prev * STAGE_ROWS
                st = pltpu.make_async_copy(
                    stage_vmem.at[pslot],
                    out_ref.at[pl.ds(base + prow, STAGE_ROWS), :],
                    st_sems.at[pslot],
                )
                st.start()
                sts[pslot] = st
            return True

        def stage_finish():
            while stage_issue():
                pass
            last = n_stage - 1
            lslot = last % N_SLOTS
            lds[lslot].wait()
            st = pltpu.make_async_copy(
                stage_vmem.at[lslot],
                out_ref.at[pl.ds(base + last * STAGE_ROWS, STAGE_ROWS), :],
                st_sems.at[lslot],
            )
            st.start()
            sts[lslot] = st
            for s in sts:
                if s is not None:
                    s.wait()

        y_rdmas = []
        for c in range(n_chunks):
            x_rdmas[c].wait_recv()
            row = other_x * m + my_y * half + offsets[c]
            ch = ROW_CHUNKS[c]
            f = pltpu.make_async_remote_copy(
                src_ref=out_ref.at[pl.ds(row, ch), :],
                dst_ref=out_ref.at[pl.ds(row, ch), :],
                send_sem=y_send.at[c],
                recv_sem=y_recv.at[c],
                device_id=(my_x, 1 - my_y),
                device_id_type=pl.DeviceIdType.MESH,
            )
            f.start()
            y_rdmas.append(f)
            stage_issue()
        stage_finish()

        for c in range(n_chunks):
            y_rdmas[c].wait_recv()
        for c in range(n_chunks):
            x_rdmas[c].wait_send()
            y_rdmas[c].wait_send()

    return pl.pallas_call(
        body,
        out_shape=jax.ShapeDtypeStruct((2 * m, n), x.dtype),
        in_specs=[pl.BlockSpec(memory_space=pl.ANY)],
        out_specs=pl.BlockSpec(memory_space=pl.ANY),
        scratch_shapes=[
            pltpu.VMEM((N_SLOTS, STAGE_ROWS, n), x.dtype),
            pltpu.SemaphoreType.DMA((N_SLOTS,)),
            pltpu.SemaphoreType.DMA((N_SLOTS,)),
            pltpu.SemaphoreType.DMA((n_chunks,)),
            pltpu.SemaphoreType.DMA((n_chunks,)),
            pltpu.SemaphoreType.DMA((n_chunks,)),
            pltpu.SemaphoreType.DMA((n_chunks,)),
        ],
    )(x)


# device time: 921280 ns/iter; 1.0892x vs baseline; 1.0039x over previous
import jax
import jax.numpy as jnp
from jax import lax
from jax.experimental import pallas as pl
from jax.experimental.pallas import tpu as pltpu

ROW_CHUNKS = [128] * 4 + [512] * 30 + [128] * 4
STAGE_ROWS = 2048
N_SLOTS = 4


def kernel(x):
    m, n = x.shape
    half = m // 2
    n_chunks = len(ROW_CHUNKS)
    assert sum(ROW_CHUNKS) == half
    offsets = [sum(ROW_CHUNKS[:i]) for i in range(n_chunks)]
    n_stage = m // STAGE_ROWS

    def body(x_ref, out_ref, stage_vmem, ld_sems, st_sems,
             x_send, x_recv, y_send, y_recv):
        my_x = lax.axis_index("x")
        my_y = lax.axis_index("y")
        other_x = 1 - my_x
        base = my_x * m

        barrier_sem = pltpu.get_barrier_semaphore()
        for nbr in [(other_x, my_y), (my_x, 1 - my_y)]:
            pl.semaphore_signal(
                barrier_sem, inc=1, device_id=nbr,
                device_id_type=pl.DeviceIdType.MESH)
        pl.semaphore_wait(barrier_sem, 2)

        x_rdmas = []
        for c in range(n_chunks):
            row = my_y * half + offsets[c]
            ch = ROW_CHUNKS[c]
            r = pltpu.make_async_remote_copy(
                src_ref=x_ref.at[pl.ds(row, ch), :],
                dst_ref=out_ref.at[pl.ds(base + row, ch), :],
                send_sem=x_send.at[c],
                recv_sem=x_recv.at[c],
                device_id=(other_x, my_y),
                device_id_type=pl.DeviceIdType.MESH,
            )
            r.start()
            x_rdmas.append(r)

        lds = [None] * N_SLOTS
        sts = [None] * N_SLOTS
        issued = [0]

        def stage_issue():
            c = issued[0]
            if c >= n_stage:
                return False
            issued[0] = c + 1
            slot = c % N_SLOTS
            if sts[slot] is not None:
                sts[slot].wait()
            row = c * STAGE_ROWS
            ld = pltpu.make_async_copy(
                x_ref.at[pl.ds(row, STAGE_ROWS), :],
                stage_vmem.at[slot],
                ld_sems.at[slot],
            )
            ld.start()
            lds[slot] = ld
            prev = c - 1
            if prev >= 0:
                pslot = prev % N_SLOTS
                lds[pslot].wait()
                prow = prev * STAGE_ROWS
                st = pltpu.make_async_copy(
                    stage_vmem.at[pslot],
                    out_ref.at[pl.ds(base + prow, STAGE_ROWS), :],
                    st_sems.at[pslot],
                )
                st.start()
                sts[pslot] = st
            return True

        def stage_finish():
            while stage_issue():
                pass
            last = n_stage - 1
            lslot = last % N_SLOTS
            lds[lslot].wait()
            st = pltpu.make_async_copy(
                stage_vmem.at[lslot],
                out_ref.at[pl.ds(base + last * STAGE_ROWS, STAGE_ROWS), :],
                st_sems.at[lslot],
            )
            st.start()
            sts[lslot] = st
            for s in sts:
                if s is not None:
                    s.wait()

        y_rdmas = []
        for c in range(n_chunks):
            x_rdmas[c].wait_recv()
            row = other_x * m + my_y * half + offsets[c]
            ch = ROW_CHUNKS[c]
            f = pltpu.make_async_remote_copy(
                src_ref=out_ref.at[pl.ds(row, ch), :],
                dst_ref=out_ref.at[pl.ds(row, ch), :],
                send_sem=y_send.at[c],
                recv_sem=y_recv.at[c],
                device_id=(my_x, 1 - my_y),
                device_id_type=pl.DeviceIdType.MESH,
            )
            f.start()
            y_rdmas.append(f)
            stage_issue()
        stage_finish()

        for c in range(n_chunks):
            y_rdmas[c].wait_recv()
        for c in range(n_chunks):
            x_rdmas[c].wait_send()
            y_rdmas[c].wait_send()

    return pl.pallas_call(
        body,
        out_shape=jax.ShapeDtypeStruct((2 * m, n), x.dtype),
        in_specs=[pl.BlockSpec(memory_space=pl.ANY)],
        out_specs=pl.BlockSpec(memory_space=pl.ANY),
        scratch_shapes=[
            pltpu.VMEM((N_SLOTS, STAGE_ROWS, n), x.dtype),
            pltpu.SemaphoreType.DMA((N_SLOTS,)),
            pltpu.SemaphoreType.DMA((N_SLOTS,)),
            pltpu.SemaphoreType.DMA((n_chunks,)),
            pltpu.SemaphoreType.DMA((n_chunks,)),
            pltpu.SemaphoreType.DMA((n_chunks,)),
            pltpu.SemaphoreType.DMA((n_chunks,)),
        ],
        compiler_params=pltpu.CompilerParams(collective_id=0),
    )(x)
